# baseline (speedup 1.0000x reference)
import math

import jax
import jax.numpy as jnp
from jax import lax
from jax.experimental import pallas as pl
from jax.experimental.pallas import tpu as pltpu
from jax.experimental.pallas import tpu_sc as plsc

D_MODEL = 128
SCALE = math.sqrt(128.0)
NUM_CORES = 2
NUM_SUBCORES = 16
NUM_WORKERS = NUM_CORES * NUM_SUBCORES
CHUNK = 64
NBUF = 4
LANES = 16


def _emb_body(x_hbm, table_hbm, out_hbm, idx_v, rows_v, outs_v, *sems):
    wid = lax.axis_index("s") * NUM_CORES + lax.axis_index("c")
    chunks = x_hbm.shape[1]
    groups = chunks // NBUF
    base = wid * (chunks * CHUNK)
    gsems = sems[:NBUF]
    osems = sems[NBUF:]

    def gather_start(j, b):
        pltpu.async_copy(table_hbm.at[idx_v.at[j]], rows_v.at[b], gsems[b])

    def gather_wait(b):
        pltpu.make_async_copy(
            table_hbm.at[idx_v.at[0]], rows_v.at[b], gsems[b]).wait()

    def out_start(j, b):
        pltpu.async_copy(outs_v.at[b],
                         out_hbm.at[pl.ds(base + j * CHUNK, CHUNK)], osems[b])

    def out_wait(b):
        pltpu.make_async_copy(
            outs_v.at[b], out_hbm.at[pl.ds(base, CHUNK)], osems[b]).wait()

    def scale(b):
        def row(r, _):
            for c in range(D_MODEL // LANES):
                sl = pl.ds(c * LANES, LANES)
                outs_v[b, r, sl] = rows_v[b, r, sl] * SCALE
            return ()

        lax.fori_loop(0, CHUNK, row, ())

    def group(g, first, last):
        for b in range(NBUF):
            j = g * NBUF + b
            gather_wait(b)
            if not first:
                out_wait(b)
            scale(b)
            out_start(j, b)
            if not last:
                gather_start(j + NBUF, b)

    pltpu.sync_copy(x_hbm.at[wid], idx_v)
    for b in range(NBUF):
        gather_start(b, b)

    group(0, first=True, last=False)

    def mid(g, _):
        group(g, first=False, last=False)
        return ()

    lax.fori_loop(1, groups - 1, mid, ())
    group(groups - 1, first=False, last=True)

    for b in range(NBUF):
        out_wait(b)


def kernel(x, table):
    b, s = x.shape
    total = b * s
    per_w = total // NUM_WORKERS
    chunks = per_w // CHUNK
    xt = jnp.swapaxes(x.astype(jnp.int32), 0, 1)
    x3 = xt.reshape(NUM_WORKERS, chunks, CHUNK)

    run = pl.kernel(
        _emb_body,
        out_type=jax.ShapeDtypeStruct((total, D_MODEL), jnp.float32),
        mesh=plsc.VectorSubcoreMesh(core_axis_name="c", subcore_axis_name="s"),
        scratch_types=[
            pltpu.VMEM((chunks, CHUNK), jnp.int32),
            pltpu.VMEM((NBUF, CHUNK, D_MODEL), jnp.float32),
            pltpu.VMEM((NBUF, CHUNK, D_MODEL), jnp.float32),
        ] + [pltpu.SemaphoreType.DMA] * (2 * NBUF),
    )
    out = run(x3, table)
    return jnp.swapaxes(out.reshape(s, b, D_MODEL), 0, 1)
